# baseline (speedup 1.0000x reference)
import functools

import jax
import jax.numpy as jnp
from jax import lax
from jax.experimental import pallas as pl
from jax.experimental.pallas import tpu as pltpu
from jax.experimental.pallas import tpu_sc as plsc

B = 16384
EDIM = 16
NT = 4
NC, NS = 2, 16
NW = NC * NS
ROWS_PER_W = B // NW
CHUNK = 128
NCHUNK = ROWS_PER_W // CHUNK
H1 = 256
H2 = 128
BN_EPS_ = 1e-5


def _sc_gather(idx_r, e0, e1, e2, e3):
    mesh = plsc.VectorSubcoreMesh(core_axis_name="c", subcore_axis_name="s")

    @functools.partial(
        pl.kernel,
        mesh=mesh,
        compiler_params=pltpu.CompilerParams(use_tc_tiling_on_sc=False,
                                             needs_layout_passes=False),
        out_type=jax.ShapeDtypeStruct((NW, NT, EDIM, ROWS_PER_W), jnp.float32),
        scratch_types=[
            pltpu.VMEM((NT * NCHUNK, CHUNK), jnp.int32),
            pltpu.VMEM((2, ROWS_PER_W, EDIM), jnp.float32),
            pltpu.VMEM((NT, EDIM, ROWS_PER_W), jnp.float32),
            pltpu.SemaphoreType.DMA,
            pltpu.SemaphoreType.DMA,
        ],
    )
    def k(idx_hbm, t0, t1, t2, t3, out_hbm, idx_v, rows_v, xgt_v, sem0, sem1):
        wid = lax.axis_index("s") * NC + lax.axis_index("c")
        pltpu.sync_copy(idx_hbm.at[wid], idx_v)
        tabs = (t0, t1, t2, t3)
        sems = (sem0, sem1)

        def fire(t):
            buf = t % 2
            return [pltpu.async_copy(
                tabs[t].at[idx_v.at[NCHUNK * t + j]],
                rows_v.at[buf, pl.ds(j * CHUNK, CHUNK)],
                sems[buf]) for j in range(NCHUNK)]

        pend = fire(0)
        for t in range(NT):
            for cp in pend:
                cp.wait()
            if t + 1 < NT:
                pend = fire(t + 1)
            buf = t % 2
            for k2 in range(ROWS_PER_W // 16):
                rows = k2 * 16 + lax.iota(jnp.int32, 16)
                for e in range(EDIM):
                    ev = jnp.full((16,), e, jnp.int32)
                    val = plsc.load_gather(rows_v.at[buf], [rows, ev])
                    xgt_v[t, e, pl.ds(k2 * 16, 16)] = val
        pltpu.sync_copy(xgt_v, out_hbm.at[wid])

    return k(idx_r, e0, e1, e2, e3)


def _tc_mlp(xg, xcont, W1, b1, W2, b2, gamma, beta):
    BLK = ROWS_PER_W

    def body(xg_ref, xc_ref, w1_ref, b1_ref, w2_ref, b2_ref, g_ref, bt_ref,
             out_ref):
        inv = 1.0 / (1.0 + BN_EPS_) ** 0.5
        x2 = xc_ref[...] * (g_ref[0, 0] * inv) + bt_ref[0, 0]
        h = x2 * w1_ref[:, EDIM * NT:EDIM * NT + 1].T + b1_ref[...]
        for t in range(NT):
            h = h + lax.dot_general(
                xg_ref[0, t], w1_ref[:, t * EDIM:(t + 1) * EDIM],
                (((0,), (1,)), ((), ())),
                preferred_element_type=jnp.float32,
                precision=lax.Precision.HIGHEST)
        out_ref[...] = lax.dot_general(
            h, w2_ref[...], (((1,), (1,)), ((), ())),
            preferred_element_type=jnp.float32,
            precision=lax.Precision.HIGHEST) + b2_ref[...]

    return pl.pallas_call(
        body,
        grid=(NW,),
        in_specs=[
            pl.BlockSpec((1, NT, EDIM, BLK), lambda i: (i, 0, 0, 0)),
            pl.BlockSpec((BLK, 1), lambda i: (i, 0)),
            pl.BlockSpec((H1, EDIM * NT + 1), lambda i: (0, 0)),
            pl.BlockSpec((1, H1), lambda i: (0, 0)),
            pl.BlockSpec((H2, H1), lambda i: (0, 0)),
            pl.BlockSpec((1, H2), lambda i: (0, 0)),
            pl.BlockSpec((1, 1), lambda i: (0, 0)),
            pl.BlockSpec((1, 1), lambda i: (0, 0)),
        ],
        out_specs=pl.BlockSpec((BLK, H2), lambda i: (i, 0)),
        out_shape=jax.ShapeDtypeStruct((B, H2), jnp.float32),
    )(xg, xcont, W1, b1, W2, b2, gamma, beta)


def kernel(x_categorical, x_continuous, emb0, emb1, emb2, emb3,
           W1, b1, W2, b2, gamma, beta):
    idx_r = (x_categorical.astype(jnp.int32)
             .reshape(NW, ROWS_PER_W, NT)
             .swapaxes(1, 2)
             .reshape(NW, NT * NCHUNK, CHUNK))
    xg = _sc_gather(idx_r, emb0, emb1, emb2, emb3)
    out = _tc_mlp(xg, x_continuous, W1, b1.reshape(1, H1), W2,
                  b2.reshape(1, H2), gamma.reshape(1, 1), beta.reshape(1, 1))
    return out


# speedup vs baseline: 1.0377x; 1.0377x over previous
import functools

import jax
import jax.numpy as jnp
from jax import lax
from jax.experimental import pallas as pl
from jax.experimental.pallas import tpu as pltpu
from jax.experimental.pallas import tpu_sc as plsc

B = 16384
VOCAB_ = 1000000
EDIM = 16
NT = 4
GRP = 8
NC, NS = 2, 16
NW = NC * NS
ROWS_PER_W = B // NW
CHUNK = 128
NCHUNK = ROWS_PER_W // CHUNK
H1 = 256
H2 = 128
BN_EPS_ = 1e-5


def _sc_gather(idx8_r, sub_r, e0, e1, e2, e3):
    mesh = plsc.VectorSubcoreMesh(core_axis_name="c", subcore_axis_name="s")

    @functools.partial(
        pl.kernel,
        mesh=mesh,
        compiler_params=pltpu.CompilerParams(needs_layout_passes=False),
        out_type=jax.ShapeDtypeStruct((NW, NT, EDIM, ROWS_PER_W), jnp.float32),
        scratch_types=[
            pltpu.VMEM((NT * NCHUNK, CHUNK), jnp.int32),
            pltpu.VMEM((NT * NCHUNK, CHUNK), jnp.int32),
            pltpu.VMEM((ROWS_PER_W, CHUNK), jnp.float32),
            pltpu.VMEM((NT, EDIM, ROWS_PER_W), jnp.float32),
            pltpu.SemaphoreType.DMA,
        ],
    )
    def k(idx8_hbm, sub_hbm, t0, t1, t2, t3, out_hbm,
          idx_v, sub_v, grp_v, xgt_v, sem):
        wid = lax.axis_index("s") * NC + lax.axis_index("c")
        pltpu.sync_copy(idx8_hbm.at[wid], idx_v)
        pltpu.sync_copy(sub_hbm.at[wid], sub_v)
        for t, tab in enumerate((t0, t1, t2, t3)):
            cps = [pltpu.async_copy(
                tab.at[idx_v.at[NCHUNK * t + j]],
                grp_v.at[pl.ds(j * CHUNK, CHUNK)],
                sem) for j in range(NCHUNK)]
            for cp in cps:
                cp.wait()
            for k2 in range(ROWS_PER_W // 16):
                rows = k2 * 16 + lax.iota(jnp.int32, 16)
                sub16 = sub_v[NCHUNK * t + k2 // 8, pl.ds((k2 % 8) * 16, 16)]
                lane0 = sub16 * EDIM
                for e in range(EDIM):
                    val = plsc.load_gather(grp_v, [rows, lane0 + e])
                    xgt_v[t, e, pl.ds(k2 * 16, 16)] = val
        pltpu.sync_copy(xgt_v, out_hbm.at[wid])

    return k(idx8_r, sub_r, e0, e1, e2, e3)


def _tc_mlp(xg, xcont, W1, b1, W2, b2, gamma, beta):
    BLK = ROWS_PER_W

    def body(xg_ref, xc_ref, w1_ref, b1_ref, w2_ref, b2_ref, g_ref, bt_ref,
             out_ref):
        inv = 1.0 / (1.0 + BN_EPS_) ** 0.5
        x2 = xc_ref[...] * (g_ref[0, 0] * inv) + bt_ref[0, 0]
        h = x2 * w1_ref[:, EDIM * NT:EDIM * NT + 1].T + b1_ref[...]
        for t in range(NT):
            h = h + lax.dot_general(
                xg_ref[0, t], w1_ref[:, t * EDIM:(t + 1) * EDIM],
                (((0,), (1,)), ((), ())),
                preferred_element_type=jnp.float32)
        out_ref[...] = lax.dot_general(
            h, w2_ref[...], (((1,), (1,)), ((), ())),
            preferred_element_type=jnp.float32) + b2_ref[...]

    return pl.pallas_call(
        body,
        grid=(NW,),
        in_specs=[
            pl.BlockSpec((1, NT, EDIM, BLK), lambda i: (i, 0, 0, 0)),
            pl.BlockSpec((BLK, 1), lambda i: (i, 0)),
            pl.BlockSpec((H1, EDIM * NT + 1), lambda i: (0, 0)),
            pl.BlockSpec((1, H1), lambda i: (0, 0)),
            pl.BlockSpec((H2, H1), lambda i: (0, 0)),
            pl.BlockSpec((1, H2), lambda i: (0, 0)),
            pl.BlockSpec((1, 1), lambda i: (0, 0)),
            pl.BlockSpec((1, 1), lambda i: (0, 0)),
        ],
        out_specs=pl.BlockSpec((BLK, H2), lambda i: (i, 0)),
        out_shape=jax.ShapeDtypeStruct((B, H2), jnp.float32),
    )(xg, xcont, W1, b1, W2, b2, gamma, beta)


def kernel(x_categorical, x_continuous, emb0, emb1, emb2, emb3,
           W1, b1, W2, b2, gamma, beta):
    xi = x_categorical.astype(jnp.int32)
    idx_r = (xi.reshape(NW, ROWS_PER_W, NT)
             .swapaxes(1, 2)
             .reshape(NW, NT * NCHUNK, CHUNK))
    idx8_r = idx_r >> 3
    sub_r = idx_r & 7
    tabs = [e.reshape(VOCAB_ // GRP, GRP * EDIM) for e in (emb0, emb1, emb2, emb3)]
    xg = _sc_gather(idx8_r, sub_r, *tabs)
    out = _tc_mlp(xg, x_continuous, W1, b1.reshape(1, H1), W2,
                  b2.reshape(1, H2), gamma.reshape(1, 1), beta.reshape(1, 1))
    return out


# speedup vs baseline: 1.2207x; 1.1763x over previous
import functools

import jax
import jax.numpy as jnp
from jax import lax
from jax.experimental import pallas as pl
from jax.experimental.pallas import tpu as pltpu
from jax.experimental.pallas import tpu_sc as plsc

B = 16384
VOCAB_ = 1000000
EDIM = 16
NT = 4
GRP = 8
NC, NS = 2, 16
NW = NC * NS
ROWS_PER_W = B // NW
CHUNK = 128
NCHUNK = ROWS_PER_W // CHUNK
H1 = 256
H2 = 128
BN_EPS_ = 1e-5


def _sc_gather(idx8_r, sub_r, e0, e1, e2, e3):
    mesh = plsc.VectorSubcoreMesh(core_axis_name="c", subcore_axis_name="s")

    @functools.partial(
        pl.kernel,
        mesh=mesh,
        compiler_params=pltpu.CompilerParams(needs_layout_passes=False),
        out_type=jax.ShapeDtypeStruct((NW, NT, EDIM, ROWS_PER_W), jnp.float32),
        scratch_types=[
            pltpu.VMEM((NT * NCHUNK, CHUNK), jnp.int32),
            pltpu.VMEM((NT * NCHUNK, CHUNK), jnp.int32),
            pltpu.VMEM((ROWS_PER_W, CHUNK), jnp.float32),
            pltpu.VMEM((NT, EDIM, ROWS_PER_W), jnp.float32),
            pltpu.SemaphoreType.DMA,
        ],
    )
    def k(idx8_hbm, sub_hbm, t0, t1, t2, t3, out_hbm,
          idx_v, sub_v, grp_v, xgt_v, sem):
        wid = lax.axis_index("s") * NC + lax.axis_index("c")
        pltpu.sync_copy(idx8_hbm.at[wid], idx_v)
        pltpu.sync_copy(sub_hbm.at[wid], sub_v)
        for t, tab in enumerate((t0, t1, t2, t3)):
            cps = [pltpu.async_copy(
                tab.at[idx_v.at[NCHUNK * t + j]],
                grp_v.at[pl.ds(j * CHUNK, CHUNK)],
                sem) for j in range(NCHUNK)]
            for cp in cps:
                cp.wait()
            for k2 in range(ROWS_PER_W // 16):
                rows = k2 * 16 + lax.iota(jnp.int32, 16)
                sub16 = sub_v[NCHUNK * t + k2 // 8, pl.ds((k2 % 8) * 16, 16)]
                lane0 = sub16 * EDIM
                for e in range(EDIM):
                    val = plsc.load_gather(grp_v, [rows, lane0 + e])
                    xgt_v[t, e, pl.ds(k2 * 16, 16)] = val
        pltpu.sync_copy(xgt_v, out_hbm.at[wid])

    return k(idx8_r, sub_r, e0, e1, e2, e3)


FBLK = 16384
FGRID = -(-VOCAB_ // FBLK)


def _tc_format(embT):
    def body(x_ref, out_ref):
        t = x_ref[...].T
        t3 = t.reshape(FBLK // GRP, GRP, EDIM)
        pieces = [t3[:, s, :] for s in range(GRP)]
        out_ref[...] = jnp.concatenate(pieces, axis=1)

    return pl.pallas_call(
        body,
        grid=(FGRID,),
        in_specs=[pl.BlockSpec((EDIM, FBLK), lambda i: (0, i))],
        out_specs=pl.BlockSpec((FBLK // GRP, GRP * EDIM), lambda i: (i, 0)),
        out_shape=jax.ShapeDtypeStruct((VOCAB_ // GRP, GRP * EDIM),
                                       jnp.float32),
    )(embT)


def _tc_mlp(xg, xcont, W1, b1, W2, b2, gamma, beta):
    BLK = ROWS_PER_W

    def body(xg_ref, xc_ref, w1_ref, b1_ref, w2_ref, b2_ref, g_ref, bt_ref,
             out_ref):
        inv = 1.0 / (1.0 + BN_EPS_) ** 0.5
        x2 = xc_ref[...] * (g_ref[0, 0] * inv) + bt_ref[0, 0]
        h = x2 * w1_ref[:, EDIM * NT:EDIM * NT + 1].T + b1_ref[...]
        for t in range(NT):
            h = h + lax.dot_general(
                xg_ref[0, t], w1_ref[:, t * EDIM:(t + 1) * EDIM],
                (((0,), (1,)), ((), ())),
                preferred_element_type=jnp.float32)
        out_ref[...] = lax.dot_general(
            h, w2_ref[...], (((1,), (1,)), ((), ())),
            preferred_element_type=jnp.float32) + b2_ref[...]

    return pl.pallas_call(
        body,
        grid=(NW,),
        in_specs=[
            pl.BlockSpec((1, NT, EDIM, BLK), lambda i: (i, 0, 0, 0)),
            pl.BlockSpec((BLK, 1), lambda i: (i, 0)),
            pl.BlockSpec((H1, EDIM * NT + 1), lambda i: (0, 0)),
            pl.BlockSpec((1, H1), lambda i: (0, 0)),
            pl.BlockSpec((H2, H1), lambda i: (0, 0)),
            pl.BlockSpec((1, H2), lambda i: (0, 0)),
            pl.BlockSpec((1, 1), lambda i: (0, 0)),
            pl.BlockSpec((1, 1), lambda i: (0, 0)),
        ],
        out_specs=pl.BlockSpec((BLK, H2), lambda i: (i, 0)),
        out_shape=jax.ShapeDtypeStruct((B, H2), jnp.float32),
    )(xg, xcont, W1, b1, W2, b2, gamma, beta)


def kernel(x_categorical, x_continuous, emb0, emb1, emb2, emb3,
           W1, b1, W2, b2, gamma, beta):
    xi = x_categorical.astype(jnp.int32)
    idx_r = (xi.reshape(NW, ROWS_PER_W, NT)
             .swapaxes(1, 2)
             .reshape(NW, NT * NCHUNK, CHUNK))
    idx8_r = idx_r >> 3
    sub_r = idx_r & 7
    tabs = [_tc_format(e.T) for e in (emb0, emb1, emb2, emb3)]
    xg = _sc_gather(idx8_r, sub_r, *tabs)
    out = _tc_mlp(xg, x_continuous, W1, b1.reshape(1, H1), W2,
                  b2.reshape(1, H2), gamma.reshape(1, 1), beta.reshape(1, 1))
    return out


# speedup vs baseline: 4.3110x; 3.5317x over previous
import functools

import jax
import jax.numpy as jnp
from jax import lax
from jax.experimental import pallas as pl
from jax.experimental.pallas import tpu as pltpu
from jax.experimental.pallas import tpu_sc as plsc

B = 16384
VOCAB_ = 1000000
EDIM = 16
NT = 4
GRP = 8
NC, NS = 2, 16
NW = NC * NS
ROWS_PER_W = B // NW
CHUNK = 128
NCHUNK = ROWS_PER_W // CHUNK
H1 = 256
H2 = 128
BN_EPS_ = 1e-5


def _sc_gather(idx8_r, sub_r, e0, e1, e2, e3):
    mesh = plsc.VectorSubcoreMesh(core_axis_name="c", subcore_axis_name="s")

    @functools.partial(
        pl.kernel,
        mesh=mesh,
        compiler_params=pltpu.CompilerParams(needs_layout_passes=False),
        out_type=jax.ShapeDtypeStruct((NW, NT, EDIM, ROWS_PER_W), jnp.float32),
        scratch_types=[
            pltpu.VMEM((NT * NCHUNK, CHUNK), jnp.int32),
            pltpu.VMEM((NT * NCHUNK, CHUNK), jnp.int32),
            pltpu.VMEM((ROWS_PER_W, CHUNK), jnp.float32),
            pltpu.VMEM((NT, EDIM, ROWS_PER_W), jnp.float32),
            pltpu.SemaphoreType.DMA,
        ],
    )
    def k(idx8_hbm, sub_hbm, t0, t1, t2, t3, out_hbm,
          idx_v, sub_v, grp_v, xgt_v, sem):
        wid = lax.axis_index("s") * NC + lax.axis_index("c")
        pltpu.sync_copy(idx8_hbm.at[wid], idx_v)
        pltpu.sync_copy(sub_hbm.at[wid], sub_v)
        for t, tab in enumerate((t0, t1, t2, t3)):
            cps = [pltpu.async_copy(
                tab.at[idx_v.at[NCHUNK * t + j]],
                grp_v.at[pl.ds(j * CHUNK, CHUNK)],
                sem) for j in range(NCHUNK)]
            for cp in cps:
                cp.wait()
            for k2 in range(ROWS_PER_W // 16):
                rows = k2 * 16 + lax.iota(jnp.int32, 16)
                sub16 = sub_v[NCHUNK * t + k2 // 8, pl.ds((k2 % 8) * 16, 16)]
                lane0 = sub16 * EDIM
                for e in range(EDIM):
                    val = plsc.load_gather(grp_v, [rows, lane0 + e])
                    xgt_v[t, e, pl.ds(k2 * 16, 16)] = val
        pltpu.sync_copy(xgt_v, out_hbm.at[wid])

    return k(idx8_r, sub_r, e0, e1, e2, e3)


FBLK = 16384
FGRID = -(-VOCAB_ // FBLK)
NLINE = FGRID * FBLK // GRP


def _tc_format(embT):
    def body(x_ref, out_ref):
        for c in range(FBLK // 1024):
            sq = jnp.concatenate(
                [x_ref[:, 1024 * c + 128 * p:1024 * c + 128 * (p + 1)]
                 for p in range(GRP)], axis=0)
            out_ref[128 * c:128 * (c + 1), :] = sq.T

    return pl.pallas_call(
        body,
        grid=(FGRID,),
        in_specs=[pl.BlockSpec((EDIM, FBLK), lambda i: (0, i))],
        out_specs=pl.BlockSpec((FBLK // GRP, GRP * EDIM), lambda i: (i, 0)),
        out_shape=jax.ShapeDtypeStruct((NLINE, GRP * EDIM), jnp.float32),
    )(embT)


def _tc_mlp(xg, xcont, W1, b1, W2, b2, gamma, beta):
    BLK = ROWS_PER_W

    def body(xg_ref, xc_ref, w1_ref, b1_ref, w2_ref, b2_ref, g_ref, bt_ref,
             out_ref):
        inv = 1.0 / (1.0 + BN_EPS_) ** 0.5
        x2 = xc_ref[...] * (g_ref[0, 0] * inv) + bt_ref[0, 0]
        h = x2 * w1_ref[:, EDIM * NT:EDIM * NT + 1].T + b1_ref[...]
        for t in range(NT):
            h = h + lax.dot_general(
                xg_ref[0, t], w1_ref[:, t * EDIM:(t + 1) * EDIM],
                (((0,), (1,)), ((), ())),
                preferred_element_type=jnp.float32)
        out_ref[...] = lax.dot_general(
            h, w2_ref[...], (((1,), (1,)), ((), ())),
            preferred_element_type=jnp.float32) + b2_ref[...]

    return pl.pallas_call(
        body,
        grid=(NW,),
        in_specs=[
            pl.BlockSpec((1, NT, EDIM, BLK), lambda i: (i, 0, 0, 0)),
            pl.BlockSpec((BLK, 1), lambda i: (i, 0)),
            pl.BlockSpec((H1, EDIM * NT + 1), lambda i: (0, 0)),
            pl.BlockSpec((1, H1), lambda i: (0, 0)),
            pl.BlockSpec((H2, H1), lambda i: (0, 0)),
            pl.BlockSpec((1, H2), lambda i: (0, 0)),
            pl.BlockSpec((1, 1), lambda i: (0, 0)),
            pl.BlockSpec((1, 1), lambda i: (0, 0)),
        ],
        out_specs=pl.BlockSpec((BLK, H2), lambda i: (i, 0)),
        out_shape=jax.ShapeDtypeStruct((B, H2), jnp.float32),
    )(xg, xcont, W1, b1, W2, b2, gamma, beta)


def kernel(x_categorical, x_continuous, emb0, emb1, emb2, emb3,
           W1, b1, W2, b2, gamma, beta):
    xi = x_categorical.astype(jnp.int32)
    idx_r = (xi.reshape(NW, ROWS_PER_W, NT)
             .swapaxes(1, 2)
             .reshape(NW, NT * NCHUNK, CHUNK))
    idx8_r = (idx_r >> 10) * 128 + (idx_r & 127)
    sub_r = (idx_r >> 7) & 7
    tabs = [_tc_format(e.T) for e in (emb0, emb1, emb2, emb3)]
    xg = _sc_gather(idx8_r, sub_r, *tabs)
    out = _tc_mlp(xg, x_continuous, W1, b1.reshape(1, H1), W2,
                  b2.reshape(1, H2), gamma.reshape(1, 1), beta.reshape(1, 1))
    return out


# speedup vs baseline: 6.1535x; 1.4274x over previous
import functools

import jax
import jax.numpy as jnp
from jax import lax
from jax.experimental import pallas as pl
from jax.experimental.pallas import tpu as pltpu
from jax.experimental.pallas import tpu_sc as plsc

B = 16384
VOCAB_ = 1000000
EDIM = 16
NT = 4
GRP = 8
NC, NS = 2, 16
NW = NC * NS
ROWS_PER_W = B // NW
CHUNK = 128
NCHUNK = ROWS_PER_W // CHUNK
H1 = 256
H2 = 128
BN_EPS_ = 1e-5


def _sc_gather(idx8_r, sub_r, e0, e1, e2, e3):
    mesh = plsc.VectorSubcoreMesh(core_axis_name="c", subcore_axis_name="s")

    @functools.partial(
        pl.kernel,
        mesh=mesh,
        compiler_params=pltpu.CompilerParams(needs_layout_passes=False),
        out_type=jax.ShapeDtypeStruct((NW, NT, EDIM, ROWS_PER_W), jnp.float32),
        scratch_types=[
            pltpu.VMEM((NT * NCHUNK, CHUNK), jnp.int32),
            pltpu.VMEM((NT * NCHUNK, CHUNK), jnp.int32),
            pltpu.VMEM((ROWS_PER_W, CHUNK), jnp.float32),
            pltpu.VMEM((NT, EDIM, ROWS_PER_W), jnp.float32),
            pltpu.SemaphoreType.DMA,
        ],
    )
    def k(idx8_hbm, sub_hbm, t0, t1, t2, t3, out_hbm,
          idx_v, sub_v, grp_v, xgt_v, sem):
        wid = lax.axis_index("s") * NC + lax.axis_index("c")
        pltpu.sync_copy(idx8_hbm.at[wid], idx_v)
        pltpu.sync_copy(sub_hbm.at[wid], sub_v)
        for t, tab in enumerate((t0, t1, t2, t3)):
            cps = [pltpu.async_copy(
                tab.at[idx_v.at[NCHUNK * t + j]],
                grp_v.at[pl.ds(j * CHUNK, CHUNK)],
                sem) for j in range(NCHUNK)]
            for cp in cps:
                cp.wait()
            for k2 in range(ROWS_PER_W // 16):
                rows = k2 * 16 + lax.iota(jnp.int32, 16)
                sub16 = sub_v[NCHUNK * t + k2 // 8, pl.ds((k2 % 8) * 16, 16)]
                lane0 = sub16 * EDIM
                for e in range(EDIM):
                    val = plsc.load_gather(grp_v, [rows, lane0 + e])
                    xgt_v[t, e, pl.ds(k2 * 16, 16)] = val
        pltpu.sync_copy(xgt_v, out_hbm.at[wid])

    return k(idx8_r, sub_r, e0, e1, e2, e3)


FBLK = 16384
FGRID = -(-VOCAB_ // FBLK)
NLINE = FGRID * FBLK // GRP


def _tc_format(embT):
    def body(*refs):
        x_refs, out_refs = refs[:NT], refs[NT:]
        for x_ref, out_ref in zip(x_refs, out_refs):
            for c in range(FBLK // 1024):
                sq = jnp.concatenate(
                    [x_ref[:, 1024 * c + 128 * p:1024 * c + 128 * (p + 1)]
                     for p in range(GRP)], axis=0)
                out_ref[128 * c:128 * (c + 1), :] = sq.T

    return pl.pallas_call(
        body,
        grid=(FGRID,),
        in_specs=[pl.BlockSpec((EDIM, FBLK), lambda i: (0, i))] * NT,
        out_specs=[pl.BlockSpec((FBLK // GRP, GRP * EDIM),
                                lambda i: (i, 0))] * NT,
        out_shape=[jax.ShapeDtypeStruct((NLINE, GRP * EDIM),
                                        jnp.float32)] * NT,
    )(*embT)


def _tc_mlp(xg, xcont, W1, b1, W2, b2, gamma, beta):
    BLK = ROWS_PER_W

    def body(xg_ref, xc_ref, w1_ref, b1_ref, w2_ref, b2_ref, g_ref, bt_ref,
             out_ref):
        inv = 1.0 / (1.0 + BN_EPS_) ** 0.5
        x2 = xc_ref[...] * (g_ref[0, 0] * inv) + bt_ref[0, 0]
        h = x2 * w1_ref[:, EDIM * NT:EDIM * NT + 1].T + b1_ref[...]
        for t in range(NT):
            h = h + lax.dot_general(
                xg_ref[0, t], w1_ref[:, t * EDIM:(t + 1) * EDIM],
                (((0,), (1,)), ((), ())),
                preferred_element_type=jnp.float32)
        out_ref[...] = lax.dot_general(
            h, w2_ref[...], (((1,), (1,)), ((), ())),
            preferred_element_type=jnp.float32) + b2_ref[...]

    return pl.pallas_call(
        body,
        grid=(NW,),
        in_specs=[
            pl.BlockSpec((1, NT, EDIM, BLK), lambda i: (i, 0, 0, 0)),
            pl.BlockSpec((BLK, 1), lambda i: (i, 0)),
            pl.BlockSpec((H1, EDIM * NT + 1), lambda i: (0, 0)),
            pl.BlockSpec((1, H1), lambda i: (0, 0)),
            pl.BlockSpec((H2, H1), lambda i: (0, 0)),
            pl.BlockSpec((1, H2), lambda i: (0, 0)),
            pl.BlockSpec((1, 1), lambda i: (0, 0)),
            pl.BlockSpec((1, 1), lambda i: (0, 0)),
        ],
        out_specs=pl.BlockSpec((BLK, H2), lambda i: (i, 0)),
        out_shape=jax.ShapeDtypeStruct((B, H2), jnp.float32),
    )(xg, xcont, W1, b1, W2, b2, gamma, beta)


def kernel(x_categorical, x_continuous, emb0, emb1, emb2, emb3,
           W1, b1, W2, b2, gamma, beta):
    xi = x_categorical.astype(jnp.int32)
    idx_r = (xi.reshape(NW, ROWS_PER_W, NT)
             .swapaxes(1, 2)
             .reshape(NW, NT * NCHUNK, CHUNK))
    idx8_r = (idx_r >> 10) * 128 + (idx_r & 127)
    sub_r = (idx_r >> 7) & 7
    tabs = _tc_format([e.T for e in (emb0, emb1, emb2, emb3)])
    xg = _sc_gather(idx8_r, sub_r, *tabs)
    out = _tc_mlp(xg, x_continuous, W1, b1.reshape(1, H1), W2,
                  b2.reshape(1, H2), gamma.reshape(1, 1), beta.reshape(1, 1))
    return out


# speedup vs baseline: 6.3051x; 1.0246x over previous
import functools

import jax
import jax.numpy as jnp
from jax import lax
from jax.experimental import pallas as pl
from jax.experimental.pallas import tpu as pltpu
from jax.experimental.pallas import tpu_sc as plsc

B = 16384
VOCAB_ = 1000000
EDIM = 16
NT = 4
GRP = 8
NC, NS = 2, 16
NW = NC * NS
ROWS_PER_W = B // NW
CHUNK = 128
NCHUNK = ROWS_PER_W // CHUNK
H1 = 256
H2 = 128
BN_EPS_ = 1e-5


def _sc_gather(idx8_r, sub_r, e0, e1, e2, e3):
    mesh = plsc.VectorSubcoreMesh(core_axis_name="c", subcore_axis_name="s")

    @functools.partial(
        pl.kernel,
        mesh=mesh,
        compiler_params=pltpu.CompilerParams(needs_layout_passes=False),
        out_type=jax.ShapeDtypeStruct((NW, NT, EDIM, ROWS_PER_W), jnp.float32),
        scratch_types=[
            pltpu.VMEM((NT * NCHUNK, CHUNK), jnp.int32),
            pltpu.VMEM((NT * NCHUNK, CHUNK), jnp.int32),
            pltpu.VMEM((ROWS_PER_W, CHUNK), jnp.float32),
            pltpu.VMEM((NT, EDIM, ROWS_PER_W), jnp.float32),
            pltpu.SemaphoreType.DMA,
        ],
    )
    def k(idx8_hbm, sub_hbm, t0, t1, t2, t3, out_hbm,
          idx_v, sub_v, grp_v, xgt_v, sem):
        wid = lax.axis_index("s") * NC + lax.axis_index("c")
        pltpu.sync_copy(idx8_hbm.at[wid], idx_v)
        pltpu.sync_copy(sub_hbm.at[wid], sub_v)
        for t, tab in enumerate((t0, t1, t2, t3)):
            cps = [pltpu.async_copy(
                tab.at[idx_v.at[NCHUNK * t + j]],
                grp_v.at[pl.ds(j * CHUNK, CHUNK)],
                sem) for j in range(NCHUNK)]
            for cp in cps:
                cp.wait()
            for k2 in range(ROWS_PER_W // 16):
                rows = k2 * 16 + lax.iota(jnp.int32, 16)
                sub16 = sub_v[NCHUNK * t + k2 // 8, pl.ds((k2 % 8) * 16, 16)]
                lane0 = sub16 * EDIM
                for e in range(EDIM):
                    val = plsc.load_gather(grp_v, [rows, lane0 + e])
                    xgt_v[t, e, pl.ds(k2 * 16, 16)] = val
        pltpu.sync_copy(xgt_v, out_hbm.at[wid])

    return k(idx8_r, sub_r, e0, e1, e2, e3)


FBLK = 32768
FGRID = -(-VOCAB_ // FBLK)
NLINE = FGRID * FBLK // GRP


def _tc_format(embT):
    def body(*refs):
        x_refs, out_refs = refs[:NT], refs[NT:]
        for x_ref, out_ref in zip(x_refs, out_refs):
            for c in range(FBLK // 1024):
                sq = jnp.concatenate(
                    [x_ref[:, 1024 * c + 128 * p:1024 * c + 128 * (p + 1)]
                     for p in range(GRP)], axis=0)
                out_ref[128 * c:128 * (c + 1), :] = sq.T

    return pl.pallas_call(
        body,
        grid=(FGRID,),
        compiler_params=pltpu.CompilerParams(
            vmem_limit_bytes=100 * 1024 * 1024),
        in_specs=[pl.BlockSpec((EDIM, FBLK), lambda i: (0, i))] * NT,
        out_specs=[pl.BlockSpec((FBLK // GRP, GRP * EDIM),
                                lambda i: (i, 0))] * NT,
        out_shape=[jax.ShapeDtypeStruct((NLINE, GRP * EDIM),
                                        jnp.float32)] * NT,
    )(*embT)


def _tc_mlp(xg, xcont, W1, b1, W2, b2, gamma, beta):
    BLK = ROWS_PER_W

    def body(xg_ref, xc_ref, w1_ref, b1_ref, w2_ref, b2_ref, g_ref, bt_ref,
             out_ref):
        inv = 1.0 / (1.0 + BN_EPS_) ** 0.5
        x2 = xc_ref[...] * (g_ref[0, 0] * inv) + bt_ref[0, 0]
        h = x2 * w1_ref[:, EDIM * NT:EDIM * NT + 1].T + b1_ref[...]
        for t in range(NT):
            h = h + lax.dot_general(
                xg_ref[0, t], w1_ref[:, t * EDIM:(t + 1) * EDIM],
                (((0,), (1,)), ((), ())),
                preferred_element_type=jnp.float32)
        out_ref[...] = lax.dot_general(
            h, w2_ref[...], (((1,), (1,)), ((), ())),
            preferred_element_type=jnp.float32) + b2_ref[...]

    return pl.pallas_call(
        body,
        grid=(NW,),
        in_specs=[
            pl.BlockSpec((1, NT, EDIM, BLK), lambda i: (i, 0, 0, 0)),
            pl.BlockSpec((BLK, 1), lambda i: (i, 0)),
            pl.BlockSpec((H1, EDIM * NT + 1), lambda i: (0, 0)),
            pl.BlockSpec((1, H1), lambda i: (0, 0)),
            pl.BlockSpec((H2, H1), lambda i: (0, 0)),
            pl.BlockSpec((1, H2), lambda i: (0, 0)),
            pl.BlockSpec((1, 1), lambda i: (0, 0)),
            pl.BlockSpec((1, 1), lambda i: (0, 0)),
        ],
        out_specs=pl.BlockSpec((BLK, H2), lambda i: (i, 0)),
        out_shape=jax.ShapeDtypeStruct((B, H2), jnp.float32),
    )(xg, xcont, W1, b1, W2, b2, gamma, beta)


def kernel(x_categorical, x_continuous, emb0, emb1, emb2, emb3,
           W1, b1, W2, b2, gamma, beta):
    xi = x_categorical.astype(jnp.int32)
    idx_r = (xi.reshape(NW, ROWS_PER_W, NT)
             .swapaxes(1, 2)
             .reshape(NW, NT * NCHUNK, CHUNK))
    idx8_r = (idx_r >> 10) * 128 + (idx_r & 127)
    sub_r = (idx_r >> 7) & 7
    tabs = _tc_format([e.T for e in (emb0, emb1, emb2, emb3)])
    xg = _sc_gather(idx8_r, sub_r, *tabs)
    out = _tc_mlp(xg, x_continuous, W1, b1.reshape(1, H1), W2,
                  b2.reshape(1, H2), gamma.reshape(1, 1), beta.reshape(1, 1))
    return out


# speedup vs baseline: 6.7381x; 1.0687x over previous
import functools

import jax
import jax.numpy as jnp
from jax import lax
from jax.experimental import pallas as pl
from jax.experimental.pallas import tpu as pltpu
from jax.experimental.pallas import tpu_sc as plsc

B = 16384
VOCAB_ = 1000000
EDIM = 16
NT = 4
GRP = 8
NC, NS = 2, 16
NW = NC * NS
ROWS_PER_W = B // NW
CHUNK = 128
NCHUNK = ROWS_PER_W // CHUNK
H1 = 256
H2 = 128
BN_EPS_ = 1e-5


def _sc_gather(idx8_r, sub_r, e0, e1, e2, e3):
    mesh = plsc.VectorSubcoreMesh(core_axis_name="c", subcore_axis_name="s")

    @functools.partial(
        pl.kernel,
        mesh=mesh,
        compiler_params=pltpu.CompilerParams(needs_layout_passes=False),
        out_type=jax.ShapeDtypeStruct((NW, NT, EDIM, ROWS_PER_W), jnp.float32),
        scratch_types=[
            pltpu.VMEM((NT * NCHUNK, CHUNK), jnp.int32),
            pltpu.VMEM((NT * NCHUNK, CHUNK), jnp.int32),
            pltpu.VMEM((ROWS_PER_W, CHUNK), jnp.float32),
            pltpu.VMEM((NT, EDIM, ROWS_PER_W), jnp.float32),
            pltpu.SemaphoreType.DMA,
        ],
    )
    def k(idx8_hbm, sub_hbm, t0, t1, t2, t3, out_hbm,
          idx_v, sub_v, grp_v, xgt_v, sem):
        wid = lax.axis_index("s") * NC + lax.axis_index("c")
        pltpu.sync_copy(idx8_hbm.at[wid], idx_v)
        pltpu.sync_copy(sub_hbm.at[wid], sub_v)
        tabs = (t0, t1, t2, t3)

        def fire(q):
            t, j = q // NCHUNK, q % NCHUNK
            return pltpu.async_copy(
                tabs[t].at[idx_v.at[q]],
                grp_v.at[pl.ds((q % NCHUNK) * CHUNK, CHUNK)], sem)

        nq = NT * NCHUNK
        pend = [fire(q) for q in range(NCHUNK)]
        for q in range(nq):
            pend[q % NCHUNK].wait()
            t, j = q // NCHUNK, q % NCHUNK
            for k2 in range(CHUNK // 16):
                row0 = j * CHUNK + k2 * 16
                rows = (q % NCHUNK) * CHUNK + k2 * 16 + lax.iota(jnp.int32, 16)
                sub16 = sub_v[q, pl.ds(k2 * 16, 16)]
                lane0 = sub16 * EDIM
                for e in range(EDIM):
                    val = plsc.load_gather(grp_v, [rows, lane0 + e])
                    xgt_v[t, e, pl.ds(row0, 16)] = val
            if q + NCHUNK < nq:
                pend[q % NCHUNK] = fire(q + NCHUNK)
        pltpu.sync_copy(xgt_v, out_hbm.at[wid])

    return k(idx8_r, sub_r, e0, e1, e2, e3)


FBLK = 32768
FGRID = -(-VOCAB_ // FBLK)
NLINE = FGRID * FBLK // GRP


def _tc_format(embT):
    def body(*refs):
        x_refs, out_refs = refs[:NT], refs[NT:]
        for x_ref, out_ref in zip(x_refs, out_refs):
            for c in range(FBLK // 1024):
                sq = jnp.concatenate(
                    [x_ref[:, 1024 * c + 128 * p:1024 * c + 128 * (p + 1)]
                     for p in range(GRP)], axis=0)
                out_ref[128 * c:128 * (c + 1), :] = sq.T

    return pl.pallas_call(
        body,
        grid=(FGRID,),
        compiler_params=pltpu.CompilerParams(
            vmem_limit_bytes=100 * 1024 * 1024),
        in_specs=[pl.BlockSpec((EDIM, FBLK), lambda i: (0, i))] * NT,
        out_specs=[pl.BlockSpec((FBLK // GRP, GRP * EDIM),
                                lambda i: (i, 0))] * NT,
        out_shape=[jax.ShapeDtypeStruct((NLINE, GRP * EDIM),
                                        jnp.float32)] * NT,
    )(*embT)


def _tc_mlp(xg, xcont, W1, b1, W2, b2, gamma, beta):
    BLK = ROWS_PER_W

    WPB = 2

    def body(xg_ref, xc_ref, w1_ref, b1_ref, w2_ref, b2_ref, g_ref, bt_ref,
             out_ref):
        inv = 1.0 / (1.0 + BN_EPS_) ** 0.5
        for w in range(WPB):
            x2 = (xc_ref[pl.ds(w * BLK, BLK), :] * (g_ref[0, 0] * inv)
                  + bt_ref[0, 0])
            h = x2 * w1_ref[:, EDIM * NT:EDIM * NT + 1].T + b1_ref[...]
            for t in range(NT):
                h = h + lax.dot_general(
                    xg_ref[w, t], w1_ref[:, t * EDIM:(t + 1) * EDIM],
                    (((0,), (1,)), ((), ())),
                    preferred_element_type=jnp.float32)
            out_ref[pl.ds(w * BLK, BLK), :] = lax.dot_general(
                h, w2_ref[...], (((1,), (1,)), ((), ())),
                preferred_element_type=jnp.float32) + b2_ref[...]

    return pl.pallas_call(
        body,
        grid=(NW // WPB,),
        in_specs=[
            pl.BlockSpec((WPB, NT, EDIM, BLK), lambda i: (i, 0, 0, 0)),
            pl.BlockSpec((WPB * BLK, 1), lambda i: (i, 0)),
            pl.BlockSpec((H1, EDIM * NT + 1), lambda i: (0, 0)),
            pl.BlockSpec((1, H1), lambda i: (0, 0)),
            pl.BlockSpec((H2, H1), lambda i: (0, 0)),
            pl.BlockSpec((1, H2), lambda i: (0, 0)),
            pl.BlockSpec((1, 1), lambda i: (0, 0)),
            pl.BlockSpec((1, 1), lambda i: (0, 0)),
        ],
        out_specs=pl.BlockSpec((WPB * BLK, H2), lambda i: (i, 0)),
        out_shape=jax.ShapeDtypeStruct((B, H2), jnp.float32),
    )(xg, xcont, W1, b1, W2, b2, gamma, beta)


def kernel(x_categorical, x_continuous, emb0, emb1, emb2, emb3,
           W1, b1, W2, b2, gamma, beta):
    xi = x_categorical.astype(jnp.int32)
    idx_r = (xi.reshape(NW, ROWS_PER_W, NT)
             .swapaxes(1, 2)
             .reshape(NW, NT * NCHUNK, CHUNK))
    idx8_r = (idx_r >> 10) * 128 + (idx_r & 127)
    sub_r = (idx_r >> 7) & 7
    tabs = _tc_format([e.T for e in (emb0, emb1, emb2, emb3)])
    xg = _sc_gather(idx8_r, sub_r, *tabs)
    out = _tc_mlp(xg, x_continuous, W1, b1.reshape(1, H1), W2,
                  b2.reshape(1, H2), gamma.reshape(1, 1), beta.reshape(1, 1))
    return out


# speedup vs baseline: 6.7946x; 1.0084x over previous
import functools

import jax
import jax.numpy as jnp
from jax import lax
from jax.experimental import pallas as pl
from jax.experimental.pallas import tpu as pltpu
from jax.experimental.pallas import tpu_sc as plsc

B = 16384
VOCAB_ = 1000000
EDIM = 16
NT = 4
GRP = 8
NC, NS = 2, 16
NW = NC * NS
ROWS_PER_W = B // NW
CHUNK = 128
NCHUNK = ROWS_PER_W // CHUNK
H1 = 256
H2 = 128
BN_EPS_ = 1e-5


def _sc_gather(idx8_r, sub_r, tabs_in):
    nt = len(tabs_in)
    mesh = plsc.VectorSubcoreMesh(core_axis_name="c", subcore_axis_name="s")

    @functools.partial(
        pl.kernel,
        mesh=mesh,
        compiler_params=pltpu.CompilerParams(needs_layout_passes=False),
        out_type=jax.ShapeDtypeStruct((NW, nt, EDIM, ROWS_PER_W), jnp.float32),
        scratch_types=[
            pltpu.VMEM((nt * NCHUNK, CHUNK), jnp.int32),
            pltpu.VMEM((nt * NCHUNK, CHUNK), jnp.int32),
            pltpu.VMEM((NCHUNK * CHUNK, CHUNK), jnp.float32),
            pltpu.VMEM((nt, EDIM, ROWS_PER_W), jnp.float32),
            pltpu.SemaphoreType.DMA,
        ],
    )
    def k(idx8_hbm, sub_hbm, *rest):
        tabs, (out_hbm, idx_v, sub_v, grp_v, xgt_v, sem) = rest[:nt], rest[nt:]
        wid = lax.axis_index("s") * NC + lax.axis_index("c")
        pltpu.sync_copy(idx8_hbm.at[wid], idx_v)
        pltpu.sync_copy(sub_hbm.at[wid], sub_v)

        def fire(q):
            return pltpu.async_copy(
                tabs[q // NCHUNK].at[idx_v.at[q]],
                grp_v.at[pl.ds((q % NCHUNK) * CHUNK, CHUNK)], sem)

        nq = nt * NCHUNK
        pend = [fire(q) for q in range(NCHUNK)]
        for q in range(nq):
            pend[q % NCHUNK].wait()
            t, j = q // NCHUNK, q % NCHUNK
            for k2 in range(CHUNK // 16):
                row0 = j * CHUNK + k2 * 16
                rows = (q % NCHUNK) * CHUNK + k2 * 16 + lax.iota(jnp.int32, 16)
                sub16 = sub_v[q, pl.ds(k2 * 16, 16)]
                lane0 = sub16 * EDIM
                for e in range(EDIM):
                    val = plsc.load_gather(grp_v, [rows, lane0 + e])
                    xgt_v[t, e, pl.ds(row0, 16)] = val
            if q + NCHUNK < nq:
                pend[q % NCHUNK] = fire(q + NCHUNK)
        pltpu.sync_copy(xgt_v, out_hbm.at[wid])

    return k(idx8_r, sub_r, *tabs_in)


FBLK = 32768
FGRID = -(-VOCAB_ // FBLK)
NLINE = FGRID * FBLK // GRP


def _tc_format(embT):
    nt = len(embT)

    def body(*refs):
        x_refs, out_refs = refs[:nt], refs[nt:]
        for x_ref, out_ref in zip(x_refs, out_refs):
            for c in range(FBLK // 1024):
                sq = jnp.concatenate(
                    [x_ref[:, 1024 * c + 128 * p:1024 * c + 128 * (p + 1)]
                     for p in range(GRP)], axis=0)
                out_ref[128 * c:128 * (c + 1), :] = sq.T

    return pl.pallas_call(
        body,
        grid=(FGRID,),
        compiler_params=pltpu.CompilerParams(
            vmem_limit_bytes=100 * 1024 * 1024),
        in_specs=[pl.BlockSpec((EDIM, FBLK), lambda i: (0, i))] * nt,
        out_specs=[pl.BlockSpec((FBLK // GRP, GRP * EDIM),
                                lambda i: (i, 0))] * nt,
        out_shape=[jax.ShapeDtypeStruct((NLINE, GRP * EDIM),
                                        jnp.float32)] * nt,
    )(*embT)


def _tc_mlp(xg0, xg1, xcont, W1, b1, W2, b2, gamma, beta):
    BLK = ROWS_PER_W

    WPB = 2

    def body(xg0_ref, xg1_ref, xc_ref, w1_ref, b1_ref, w2_ref, b2_ref,
             g_ref, bt_ref, out_ref):
        inv = 1.0 / (1.0 + BN_EPS_) ** 0.5
        for w in range(WPB):
            x2 = (xc_ref[pl.ds(w * BLK, BLK), :] * (g_ref[0, 0] * inv)
                  + bt_ref[0, 0])
            h = x2 * w1_ref[:, EDIM * NT:EDIM * NT + 1].T + b1_ref[...]
            for t in range(NT):
                xgr = xg0_ref if t < 2 else xg1_ref
                h = h + lax.dot_general(
                    xgr[w, t % 2], w1_ref[:, t * EDIM:(t + 1) * EDIM],
                    (((0,), (1,)), ((), ())),
                    preferred_element_type=jnp.float32)
            out_ref[pl.ds(w * BLK, BLK), :] = lax.dot_general(
                h, w2_ref[...], (((1,), (1,)), ((), ())),
                preferred_element_type=jnp.float32) + b2_ref[...]

    return pl.pallas_call(
        body,
        grid=(NW // WPB,),
        in_specs=[
            pl.BlockSpec((WPB, NT // 2, EDIM, BLK), lambda i: (i, 0, 0, 0)),
            pl.BlockSpec((WPB, NT // 2, EDIM, BLK), lambda i: (i, 0, 0, 0)),
            pl.BlockSpec((WPB * BLK, 1), lambda i: (i, 0)),
            pl.BlockSpec((H1, EDIM * NT + 1), lambda i: (0, 0)),
            pl.BlockSpec((1, H1), lambda i: (0, 0)),
            pl.BlockSpec((H2, H1), lambda i: (0, 0)),
            pl.BlockSpec((1, H2), lambda i: (0, 0)),
            pl.BlockSpec((1, 1), lambda i: (0, 0)),
            pl.BlockSpec((1, 1), lambda i: (0, 0)),
        ],
        out_specs=pl.BlockSpec((WPB * BLK, H2), lambda i: (i, 0)),
        out_shape=jax.ShapeDtypeStruct((B, H2), jnp.float32),
    )(xg0, xg1, xcont, W1, b1, W2, b2, gamma, beta)


def kernel(x_categorical, x_continuous, emb0, emb1, emb2, emb3,
           W1, b1, W2, b2, gamma, beta):
    xi = x_categorical.astype(jnp.int32)
    idx_r = (xi.reshape(NW, ROWS_PER_W, NT)
             .swapaxes(1, 2)
             .reshape(NW, NT * NCHUNK, CHUNK))
    idx8_r = (idx_r >> 10) * 128 + (idx_r & 127)
    sub_r = (idx_r >> 7) & 7
    half = NT // 2 * NCHUNK
    tabs01 = _tc_format([emb0.T, emb1.T])
    xg0 = _sc_gather(idx8_r[:, :half], sub_r[:, :half], tabs01)
    tabs23 = _tc_format([emb2.T, emb3.T])
    xg1 = _sc_gather(idx8_r[:, half:], sub_r[:, half:], tabs23)
    out = _tc_mlp(xg0, xg1, x_continuous, W1, b1.reshape(1, H1), W2,
                  b2.reshape(1, H2), gamma.reshape(1, 1), beta.reshape(1, 1))
    return out
